# Initial kernel scaffold; baseline (speedup 1.0000x reference)
import jax
import jax.numpy as jnp
from jax.experimental import pallas as pl


def kernel(x, start, end, actionnes, gt_iou_map, gt_bbox, num_gt):
    raise NotImplementedError("write your pallas kernel here")


# speedup vs baseline: 2560.9348x; 2560.9348x over previous
import functools

import jax
import jax.numpy as jnp
from jax.experimental import pallas as pl
from jax.experimental.pallas import tpu as pltpu

_RES = 32
_TP = 100
_EXPAND = 0.5
_CH = 20
_NL = _CH * _RES


def _align_body(lo_ref, hi_ref, w_ref, x_ref, out_ref):
    T = x_ref.shape[2]
    lo = jnp.broadcast_to(lo_ref[0, 0], (T, _NL))
    hi = jnp.broadcast_to(hi_ref[0, 0], (T, _NL))
    w = jnp.broadcast_to(w_ref[0, 0], (T, _NL))
    t = jax.lax.broadcasted_iota(jnp.int32, (T, _NL), 0)
    wmat = jnp.where(t == lo, 1.0 - w, 0.0) + jnp.where(t == hi, w, 0.0)
    res = jax.lax.dot_general(x_ref[0], wmat, (((1,), (0,)), ((), ())),
                              preferred_element_type=jnp.float32)
    for i in range(_CH):
        out_ref[i] = res[:, i * _RES:(i + 1) * _RES]


def _align(x, lo, hi, w):
    B, C, T = x.shape
    nch = _TP // _CH
    lo_r = lo.reshape(B, nch, 1, _NL)
    hi_r = hi.reshape(B, nch, 1, _NL)
    w_r = w.reshape(B, nch, 1, _NL)
    spec_idx = pl.BlockSpec((1, 1, 1, _NL), lambda b, c: (b, c, 0, 0))
    return pl.pallas_call(
        _align_body,
        grid=(B, nch),
        in_specs=[
            spec_idx, spec_idx, spec_idx,
            pl.BlockSpec((1, C, T), lambda b, c: (b, 0, 0)),
        ],
        out_specs=pl.BlockSpec((_CH, C, _RES), lambda b, c: (b * nch + c, 0, 0)),
        out_shape=jax.ShapeDtypeStruct((B * _TP, C, _RES), jnp.float32),
        compiler_params=pltpu.CompilerParams(
            dimension_semantics=("parallel", "arbitrary")),
    )(lo_r, hi_r, w_r, x)


def kernel(x, start, end, actionnes, gt_iou_map, gt_bbox, num_gt):
    B, C, T = x.shape
    score = start[:, :, None] * end[:, None, :] * 0.5 * (
        actionnes[:, :, None] + actionnes[:, None, :])
    valid = jnp.triu(jnp.ones((T, T), dtype=jnp.float32), k=1)
    score = jnp.where(valid > 0, score, -1e9)
    _, top_idx = jax.lax.top_k(score.reshape(B, T * T), _TP)
    s_i = (top_idx // T).reshape(-1)
    e_i = (top_idx % T).reshape(-1)
    b_idx = jnp.repeat(jnp.arange(B, dtype=jnp.int32), _TP)
    s_f = s_i.astype(jnp.float32)
    e_f = e_i.astype(jnp.float32)
    anchor_coord = jnp.stack([b_idx.astype(jnp.float32), s_f, e_f], axis=1)
    ctx = (e_f - s_f) * _EXPAND
    s_exp = s_f - ctx
    e_exp = e_f + ctx
    iou = gt_iou_map[b_idx, s_i, e_i]
    samp_thr = jnp.mean(iou)
    pos_idx_st_end = (iou > samp_thr).astype(jnp.float32)
    pts = (jnp.arange(_RES, dtype=jnp.float32) + 0.5) / _RES
    coords = s_exp[:, None] + (e_exp - s_exp)[:, None] * pts[None, :]
    coords = jnp.clip(coords, 0.0, T - 1.0)
    lo = jnp.floor(coords).astype(jnp.int32)
    hi = jnp.minimum(lo + 1, T - 1)
    w = coords - lo.astype(jnp.float32)
    feat = _align(x, lo, hi, w)
    anchor_num = jnp.full((B,), _TP, dtype=jnp.int32)
    return feat, anchor_coord, anchor_num, samp_thr, pos_idx_st_end
